# baseline (speedup 1.0000x reference)
import functools

import jax
import jax.numpy as jnp
from jax import lax
from jax.experimental import pallas as pl
from jax.experimental.pallas import tpu as pltpu
from jax.experimental.pallas import tpu_sc as plsc

SEG = 16
NSEG = 128
TOPK = 12
DBLK = 16


def _tc_body(x_ref, a_ref, th_ref):
    xb = x_ref[0]
    e = xb.reshape(DBLK, NSEG, SEG)

    a_slices = []
    for s in range(DBLK):
        es = e[s]
        gram = jax.lax.dot_general(
            es, es,
            dimension_numbers=(((1,), (1,)), ((), ())),
            preferred_element_type=jnp.float32,
            precision=jax.lax.Precision.DEFAULT,
        )
        n2 = jnp.sum(es * es, axis=1)
        inv = 1.0 / jnp.sqrt(n2)
        a_slices.append(gram * (inv[:, None] * inv[None, :]))
    a = jnp.stack(a_slices, axis=0)

    ri = jax.lax.broadcasted_iota(jnp.int32, (NSEG, NSEG), 0)
    ci = jax.lax.broadcasted_iota(jnp.int32, (NSEG, NSEG), 1)
    diag = (ri == ci)[None]

    work = jnp.where(diag, -jnp.inf, a)
    for _ in range(TOPK - 2):
        m = jnp.max(work, axis=-2, keepdims=True)
        work = jnp.where(work == m, -jnp.inf, work)
    thresh = jnp.max(work, axis=-2, keepdims=True)
    thresh_col = thresh.reshape(DBLK, NSEG, 1)

    a_ref[:, 0] = a
    th_ref[:, 0] = jnp.broadcast_to(thresh_col, (DBLK, NSEG, SEG))


def _tc_stage(x):
    batch, ts_dim, ts_len = x.shape
    a, th = pl.pallas_call(
        _tc_body,
        grid=(batch, ts_dim // DBLK),
        in_specs=[
            pl.BlockSpec((1, DBLK, ts_len), lambda b, do: (b, do, 0)),
        ],
        out_specs=[
            pl.BlockSpec((DBLK, 1, NSEG, NSEG), lambda b, do: (do, b, 0, 0)),
            pl.BlockSpec((DBLK, 1, NSEG, SEG), lambda b, do: (do, b, 0, 0)),
        ],
        out_shape=[
            jax.ShapeDtypeStruct((ts_dim, batch, NSEG, NSEG), jnp.float32),
            jax.ShapeDtypeStruct((ts_dim, batch, NSEG, SEG), jnp.float32),
        ],
        compiler_params=pltpu.CompilerParams(
            dimension_semantics=("parallel", "parallel"),
        ),
    )(x)
    return a, th


def _sc_mask(a_flat, th_flat):
    n_slices = a_flat.shape[0]
    info = plsc.get_sparse_core_info()
    nw = info.num_cores * info.num_subcores
    per_w = n_slices // nw
    mesh = plsc.VectorSubcoreMesh(core_axis_name="c", subcore_axis_name="s")

    @functools.partial(
        pl.kernel, mesh=mesh,
        out_type=jax.ShapeDtypeStruct((n_slices, NSEG, NSEG), jnp.float32),
        scratch_types=[
            pltpu.VMEM((NSEG, NSEG), jnp.float32),
            pltpu.VMEM((NSEG, SEG), jnp.float32),
            pltpu.VMEM((NSEG, NSEG), jnp.float32),
        ],
    )
    def sc_kernel(a_hbm, th_hbm, out_hbm, a_v, th_v, o_v):
        wid = lax.axis_index("s") * info.num_cores + lax.axis_index("c")
        base = wid * per_w
        lane = lax.iota(jnp.int32, SEG)

        def do_slice(g, carry):
            sl = base + g
            pltpu.sync_copy(a_hbm.at[sl], a_v)
            pltpu.sync_copy(th_hbm.at[sl], th_v)

            def do_row(i, c2):
                th_vec = th_v[i, :]
                for c in range(NSEG // SEG):
                    v = a_v[i, pl.ds(c * SEG, SEG)]
                    keep = jnp.logical_or(v >= th_vec, (lane + c * SEG) == i)
                    o_v[i, pl.ds(c * SEG, SEG)] = jnp.where(keep, v, 0.0)
                return c2

            lax.fori_loop(0, NSEG, do_row, 0)
            pltpu.sync_copy(o_v, out_hbm.at[sl])
            return carry

        lax.fori_loop(0, per_w, do_slice, 0)

    return sc_kernel(a_flat, th_flat)


def kernel(x):
    batch, ts_dim, ts_len = x.shape
    a, th = _tc_stage(x)
    a_flat = a.reshape(ts_dim * batch, NSEG, NSEG)
    th_flat = th.reshape(ts_dim * batch, NSEG, SEG)
    out = _sc_mask(a_flat, th_flat)
    return out.reshape(ts_dim, batch, NSEG, NSEG)


# speedup vs baseline: 2.8990x; 2.8990x over previous
import jax
import jax.numpy as jnp
from jax import lax
from jax.experimental import pallas as pl
from jax.experimental.pallas import tpu as pltpu

SEG = 16
NSEG = 128
TOPK = 12
DBLK = 16


def _body(x_ref, o_ref):
    xb = x_ref[0]
    e = xb.reshape(DBLK, NSEG, SEG)

    a_slices = []
    for s in range(DBLK):
        es = e[s]
        gram = jax.lax.dot_general(
            es, es,
            dimension_numbers=(((1,), (1,)), ((), ())),
            preferred_element_type=jnp.float32,
            precision=jax.lax.Precision.DEFAULT,
        )
        n2 = jnp.sum(es * es, axis=1)
        inv = 1.0 / jnp.sqrt(n2)
        a_slices.append(gram * (inv[:, None] * inv[None, :]))
    a = jnp.stack(a_slices, axis=0)

    ri = jax.lax.broadcasted_iota(jnp.int32, (NSEG, NSEG), 0)
    ci = jax.lax.broadcasted_iota(jnp.int32, (NSEG, NSEG), 1)
    diag = (ri == ci)[None]

    work = jnp.where(diag, -jnp.inf, a)
    for _ in range(TOPK - 2):
        m = jnp.max(work, axis=-2, keepdims=True)
        work = jnp.where(work == m, -jnp.inf, work)
    thresh = jnp.max(work, axis=-2, keepdims=True)
    thresh_col = thresh.reshape(DBLK, NSEG, 1)

    keep = jnp.logical_or(a >= thresh_col, diag)
    o_ref[:, 0] = jnp.where(keep, a, 0.0)


def kernel(x):
    batch, ts_dim, ts_len = x.shape
    out = pl.pallas_call(
        _body,
        grid=(batch, ts_dim // DBLK),
        in_specs=[
            pl.BlockSpec((1, DBLK, ts_len), lambda b, do: (b, do, 0)),
        ],
        out_specs=pl.BlockSpec(
            (DBLK, 1, NSEG, NSEG), lambda b, do: (do, b, 0, 0)),
        out_shape=jax.ShapeDtypeStruct(
            (ts_dim, batch, NSEG, NSEG), jnp.float32),
        compiler_params=pltpu.CompilerParams(
            dimension_semantics=("parallel", "parallel"),
        ),
    )(x)
    return out


# speedup vs baseline: 2.9037x; 1.0016x over previous
import jax
import jax.numpy as jnp
from jax import lax
from jax.experimental import pallas as pl
from jax.experimental.pallas import tpu as pltpu

SEG = 16
NSEG = 128
TOPK = 12
DBLK = 16


def _body(x_ref, o_ref):
    xb = x_ref[0]
    e = xb.reshape(DBLK, NSEG, SEG)

    a_slices = []
    for s in range(DBLK):
        es = e[s]
        gram = jax.lax.dot_general(
            es, es,
            dimension_numbers=(((1,), (1,)), ((), ())),
            preferred_element_type=jnp.float32,
            precision=jax.lax.Precision.DEFAULT,
        )
        n2 = jnp.sum(es * es, axis=1)
        inv = 1.0 / jnp.sqrt(n2)
        a_slices.append(gram * (inv[:, None] * inv[None, :]))
    a = jnp.stack(a_slices, axis=0)

    ri = jax.lax.broadcasted_iota(jnp.int32, (NSEG, NSEG), 0)
    ci = jax.lax.broadcasted_iota(jnp.int32, (NSEG, NSEG), 1)
    diag = (ri == ci)[None]

    work = jnp.where(diag, -jnp.inf, a)
    for _ in range(TOPK - 2):
        m = jnp.max(work, axis=-2, keepdims=True)
        work = jnp.where(work == m, -jnp.inf, work)
    thresh = jnp.max(work, axis=-2, keepdims=True)
    thresh_col = thresh.reshape(DBLK, NSEG, 1)

    keep = jnp.logical_or(a >= thresh_col, diag)
    o_ref[:, 0] = jnp.where(keep, a, 0.0)


def kernel(x):
    batch, ts_dim, ts_len = x.shape
    out = pl.pallas_call(
        _body,
        grid=(ts_dim // DBLK, batch),
        in_specs=[
            pl.BlockSpec((1, DBLK, ts_len), lambda do, b: (b, do, 0)),
        ],
        out_specs=pl.BlockSpec(
            (DBLK, 1, NSEG, NSEG), lambda do, b: (do, b, 0, 0)),
        out_shape=jax.ShapeDtypeStruct(
            (ts_dim, batch, NSEG, NSEG), jnp.float32),
        compiler_params=pltpu.CompilerParams(
            dimension_semantics=("parallel", "parallel"),
        ),
    )(x)
    return out
